# Initial kernel scaffold; baseline (speedup 1.0000x reference)
import functools

import jax
import jax.numpy as jnp
from jax import lax
from jax.experimental import pallas as pl
from jax.experimental.pallas import tpu as pltpu
from jax.experimental.pallas import tpu_sc as plsc

NC = 2
NS = 16
NW = NC * NS
CH = 128


def _sc_degree(dst2d, zeros16, ones16, n_sh, nr):
    zr = n_sh // NS
    mesh = plsc.VectorSubcoreMesh(core_axis_name="c", subcore_axis_name="s")

    @functools.partial(
        pl.kernel,
        out_type=jax.ShapeDtypeStruct((NC, n_sh, 16), jnp.float32),
        mesh=mesh,
        scratch_types=[
            pltpu.VMEM((nr, CH), jnp.int32),
            pltpu.VMEM((CH, 16), jnp.float32),
            pltpu.VMEM_SHARED((n_sh, 16), jnp.float32),
        ],
    )
    def deg_kernel(dst_hbm, z_hbm, ones_hbm, out_hbm, dst_v, ones_v, deg_sh):
        cid = lax.axis_index("c")
        sid = lax.axis_index("s")
        w = cid * NS + sid
        pltpu.sync_copy(z_hbm, deg_sh.at[pl.ds(sid * zr, zr)])
        pltpu.sync_copy(ones_hbm, ones_v)
        pltpu.sync_copy(dst_hbm.at[pl.ds(w * nr, nr)], dst_v)
        plsc.subcore_barrier()

        @pl.loop(0, nr)
        def _(j):
            pltpu.sync_copy(ones_v, deg_sh.at[dst_v.at[j]], add=True)

        plsc.subcore_barrier()
        pltpu.sync_copy(deg_sh.at[pl.ds(sid * zr, zr)],
                        out_hbm.at[cid].at[pl.ds(sid * zr, zr)])

    return deg_kernel(dst2d, zeros16, ones16)


def _sc_aggregate(y, src2d, dst2d, zeros128, n_sh, nr):
    d = y.shape[1]
    zr = n_sh // NS
    mesh = plsc.VectorSubcoreMesh(core_axis_name="c", subcore_axis_name="s")

    @functools.partial(
        pl.kernel,
        out_type=jax.ShapeDtypeStruct((NC, n_sh, d), jnp.float32),
        mesh=mesh,
        scratch_types=[
            pltpu.VMEM((nr, CH), jnp.int32),
            pltpu.VMEM((nr, CH), jnp.int32),
            pltpu.VMEM((CH, d), jnp.float32),
            pltpu.VMEM_SHARED((n_sh, d), jnp.float32),
            pltpu.SemaphoreType.DMA,
        ],
    )
    def agg_kernel(y_hbm, src_hbm, dst_hbm, z_hbm, out_hbm,
                   src_v, dst_v, rows_v, acc_sh, sem):
        cid = lax.axis_index("c")
        sid = lax.axis_index("s")
        w = cid * NS + sid
        pltpu.sync_copy(z_hbm, acc_sh.at[pl.ds(sid * zr, zr)])
        pltpu.sync_copy(src_hbm.at[pl.ds(w * nr, nr)], src_v)
        pltpu.sync_copy(dst_hbm.at[pl.ds(w * nr, nr)], dst_v)
        plsc.subcore_barrier()

        @pl.loop(0, nr)
        def _(j):
            pltpu.async_copy(y_hbm.at[src_v.at[j]], rows_v, sem).wait()
            pltpu.sync_copy(rows_v, acc_sh.at[dst_v.at[j]], add=True)

        plsc.subcore_barrier()
        pltpu.sync_copy(acc_sh.at[pl.ds(sid * zr, zr)],
                        out_hbm.at[cid].at[pl.ds(sid * zr, zr)])

    return agg_kernel(y, src2d, dst2d, zeros128)


def _dinv_block(dg):
    deg = dg[0, :, 0:1] + dg[1, :, 0:1] + 1.0
    return lax.rsqrt(deg)


def _tc_matmul(x, w):
    n = x.shape[0]
    bn = 2000 if n % 2000 == 0 else n

    def body(x_ref, w_ref, o_ref):
        o_ref[...] = lax.dot_general(
            x_ref[...], w_ref[...], (((1,), (0,)), ((), ())),
            precision=lax.Precision.HIGHEST,
            preferred_element_type=jnp.float32)

    return pl.pallas_call(
        body,
        grid=(n // bn,),
        in_specs=[pl.BlockSpec((bn, x.shape[1]), lambda i: (i, 0)),
                  pl.BlockSpec(w.shape, lambda i: (0, 0))],
        out_specs=pl.BlockSpec((bn, w.shape[1]), lambda i: (i, 0)),
        out_shape=jax.ShapeDtypeStruct((n, w.shape[1]), jnp.float32),
    )(x, w)


def _tc_scale(xw, deg_p, n):
    d = xw.shape[1]
    bn = 2000 if n % 2000 == 0 else n

    def body(xw_ref, dg_ref, o_ref):
        o_ref[...] = xw_ref[...] * _dinv_block(dg_ref[...])

    return pl.pallas_call(
        body,
        grid=(n // bn,),
        in_specs=[pl.BlockSpec((bn, d), lambda i: (i, 0)),
                  pl.BlockSpec((2, bn, 16), lambda i: (0, i, 0))],
        out_specs=pl.BlockSpec((bn, d), lambda i: (i, 0)),
        out_shape=jax.ShapeDtypeStruct((n, d), jnp.float32),
    )(xw, deg_p)


def _tc_mid(acc_p, y1, deg_p, b1, w2, n):
    d = y1.shape[1]
    bn = 2000 if n % 2000 == 0 else n

    def body(a_ref, y_ref, dg_ref, b_ref, w_ref, o_ref):
        dinv = _dinv_block(dg_ref[...])
        a = a_ref[0] + a_ref[1] + y_ref[...]
        h = jnp.maximum(dinv * a + b_ref[...], 0.0)
        o_ref[...] = dinv * lax.dot_general(
            h, w_ref[...], (((1,), (0,)), ((), ())),
            precision=lax.Precision.HIGHEST,
            preferred_element_type=jnp.float32)

    return pl.pallas_call(
        body,
        grid=(n // bn,),
        in_specs=[pl.BlockSpec((2, bn, d), lambda i: (0, i, 0)),
                  pl.BlockSpec((bn, d), lambda i: (i, 0)),
                  pl.BlockSpec((2, bn, 16), lambda i: (0, i, 0)),
                  pl.BlockSpec((1, d), lambda i: (0, 0)),
                  pl.BlockSpec(w2.shape, lambda i: (0, 0))],
        out_specs=pl.BlockSpec((bn, d), lambda i: (i, 0)),
        out_shape=jax.ShapeDtypeStruct((n, d), jnp.float32),
    )(acc_p, y1, deg_p, b1, w2)


def _tc_out(acc_p, y2, deg_p, b2, n):
    d = y2.shape[1]
    bn = 2000 if n % 2000 == 0 else n

    def body(a_ref, y_ref, dg_ref, b_ref, o_ref):
        dinv = _dinv_block(dg_ref[...])
        a = a_ref[0] + a_ref[1] + y_ref[...]
        o_ref[...] = dinv * a + b_ref[...]

    return pl.pallas_call(
        body,
        grid=(n // bn,),
        in_specs=[pl.BlockSpec((2, bn, d), lambda i: (0, i, 0)),
                  pl.BlockSpec((bn, d), lambda i: (i, 0)),
                  pl.BlockSpec((2, bn, 16), lambda i: (0, i, 0)),
                  pl.BlockSpec((1, d), lambda i: (0, 0))],
        out_specs=pl.BlockSpec((bn, d), lambda i: (i, 0)),
        out_shape=jax.ShapeDtypeStruct((n, d), jnp.float32),
    )(acc_p, y2, deg_p, b2)


def kernel(x, edge_index, W1, b1, W2, b2):
    n = x.shape[0]
    e = edge_index.shape[1]
    src = edge_index[0].astype(jnp.int32)
    dst = edge_index[1].astype(jnp.int32)

    epw = -(-e // (NW * CH)) * CH
    nr = epw // CH
    e_pad = epw * NW
    pad = e_pad - e
    src_p = jnp.concatenate([src, jnp.zeros((pad,), jnp.int32)])
    dst_p = jnp.concatenate([dst, jnp.full((pad,), n, jnp.int32)])
    src2d = src_p.reshape(e_pad // CH, CH)
    dst2d = dst_p.reshape(e_pad // CH, CH)

    n_sh = -(-(n + 1) // NS) * NS
    zr = n_sh // NS
    zeros16 = jnp.zeros((zr, 16), jnp.float32)
    zeros128 = jnp.zeros((zr, W1.shape[1]), jnp.float32)
    ones16 = jnp.ones((CH, 16), jnp.float32)
    b1r = b1.reshape(1, -1)
    b2r = b2.reshape(1, -1)

    deg_p = _sc_degree(dst2d, zeros16, ones16, n_sh, nr)
    xw1 = _tc_matmul(x, W1)
    y1 = _tc_scale(xw1, deg_p, n)
    acc1 = _sc_aggregate(y1, src2d, dst2d, zeros128, n_sh, nr)
    y2 = _tc_mid(acc1, y1, deg_p, b1r, W2, n)
    acc2 = _sc_aggregate(y2, src2d, dst2d, zeros128, n_sh, nr)
    return _tc_out(acc2, y2, deg_p, b2r, n)


# speedup vs baseline: 7.8577x; 7.8577x over previous
import functools

import jax
import jax.numpy as jnp
from jax import lax
from jax.experimental import pallas as pl
from jax.experimental.pallas import tpu as pltpu
from jax.experimental.pallas import tpu_sc as plsc

NC = 2
NS = 16
NW = NC * NS
CH = 128


def _sc_degree(dst2d, zeros128, ones128, n_sh, nr):
    zr = n_sh // NS
    d = ones128.shape[1]
    mesh = plsc.VectorSubcoreMesh(core_axis_name="c", subcore_axis_name="s")

    @functools.partial(
        pl.kernel,
        out_type=jax.ShapeDtypeStruct((NC, n_sh, d), jnp.float32),
        mesh=mesh,
        scratch_types=[
            pltpu.VMEM((nr, CH), jnp.int32),
            pltpu.VMEM((CH, d), jnp.float32),
            pltpu.VMEM_SHARED((n_sh, d), jnp.float32),
        ],
    )
    def deg_kernel(dst_hbm, z_hbm, ones_hbm, out_hbm, dst_v, ones_v, deg_sh):
        cid = lax.axis_index("c")
        sid = lax.axis_index("s")
        w = cid * NS + sid
        pltpu.sync_copy(z_hbm, deg_sh.at[pl.ds(sid * zr, zr)])
        pltpu.sync_copy(ones_hbm, ones_v)
        pltpu.sync_copy(dst_hbm.at[pl.ds(w * nr, nr)], dst_v)
        plsc.subcore_barrier()

        @pl.loop(0, nr)
        def _(j):
            pltpu.sync_copy(ones_v, deg_sh.at[dst_v.at[j]], add=True)

        plsc.subcore_barrier()
        pltpu.sync_copy(deg_sh.at[pl.ds(sid * zr, zr)],
                        out_hbm.at[cid].at[pl.ds(sid * zr, zr)])

    return deg_kernel(dst2d, zeros128, ones128)


def _sc_aggregate(y, src2d, dst2d, zeros128, n_sh, nr):
    d = y.shape[1]
    zr = n_sh // NS
    mesh = plsc.VectorSubcoreMesh(core_axis_name="c", subcore_axis_name="s")

    @functools.partial(
        pl.kernel,
        out_type=jax.ShapeDtypeStruct((NC, n_sh, d), jnp.float32),
        mesh=mesh,
        scratch_types=[
            pltpu.VMEM((nr, CH), jnp.int32),
            pltpu.VMEM((nr, CH), jnp.int32),
            pltpu.VMEM((CH, d), jnp.float32),
            pltpu.VMEM_SHARED((n_sh, d), jnp.float32),
            pltpu.SemaphoreType.DMA,
        ],
    )
    def agg_kernel(y_hbm, src_hbm, dst_hbm, z_hbm, out_hbm,
                   src_v, dst_v, rows_v, acc_sh, sem):
        cid = lax.axis_index("c")
        sid = lax.axis_index("s")
        w = cid * NS + sid
        pltpu.sync_copy(z_hbm, acc_sh.at[pl.ds(sid * zr, zr)])
        pltpu.sync_copy(src_hbm.at[pl.ds(w * nr, nr)], src_v)
        pltpu.sync_copy(dst_hbm.at[pl.ds(w * nr, nr)], dst_v)
        plsc.subcore_barrier()

        @pl.loop(0, nr)
        def _(j):
            pltpu.async_copy(y_hbm.at[src_v.at[j]], rows_v, sem).wait()
            pltpu.sync_copy(rows_v, acc_sh.at[dst_v.at[j]], add=True)

        plsc.subcore_barrier()
        pltpu.sync_copy(acc_sh.at[pl.ds(sid * zr, zr)],
                        out_hbm.at[cid].at[pl.ds(sid * zr, zr)])

    return agg_kernel(y, src2d, dst2d, zeros128)


def _dinv_block(dg):
    deg = dg[0] + dg[1] + 1.0
    return lax.rsqrt(deg)


def _tc_matmul(x, w):
    n = x.shape[0]
    bn = 2000 if n % 2000 == 0 else n

    def body(x_ref, w_ref, o_ref):
        o_ref[...] = lax.dot_general(
            x_ref[...], w_ref[...], (((1,), (0,)), ((), ())),
            precision=lax.Precision.HIGHEST,
            preferred_element_type=jnp.float32)

    return pl.pallas_call(
        body,
        grid=(n // bn,),
        in_specs=[pl.BlockSpec((bn, x.shape[1]), lambda i: (i, 0)),
                  pl.BlockSpec(w.shape, lambda i: (0, 0))],
        out_specs=pl.BlockSpec((bn, w.shape[1]), lambda i: (i, 0)),
        out_shape=jax.ShapeDtypeStruct((n, w.shape[1]), jnp.float32),
    )(x, w)


def _tc_scale(xw, deg_p, n):
    d = xw.shape[1]
    bn = 2000 if n % 2000 == 0 else n

    def body(xw_ref, dg_ref, o_ref):
        o_ref[...] = xw_ref[...] * _dinv_block(dg_ref[...])

    return pl.pallas_call(
        body,
        grid=(n // bn,),
        in_specs=[pl.BlockSpec((bn, d), lambda i: (i, 0)),
                  pl.BlockSpec((2, bn, d), lambda i: (0, i, 0))],
        out_specs=pl.BlockSpec((bn, d), lambda i: (i, 0)),
        out_shape=jax.ShapeDtypeStruct((n, d), jnp.float32),
    )(xw, deg_p)


def _tc_mid(acc_p, y1, deg_p, b1, w2, n):
    d = y1.shape[1]
    bn = 2000 if n % 2000 == 0 else n

    def body(a_ref, y_ref, dg_ref, b_ref, w_ref, o_ref):
        dinv = _dinv_block(dg_ref[...])
        a = a_ref[0] + a_ref[1] + y_ref[...]
        h = jnp.maximum(dinv * a + b_ref[...], 0.0)
        o_ref[...] = dinv * lax.dot_general(
            h, w_ref[...], (((1,), (0,)), ((), ())),
            precision=lax.Precision.HIGHEST,
            preferred_element_type=jnp.float32)

    return pl.pallas_call(
        body,
        grid=(n // bn,),
        in_specs=[pl.BlockSpec((2, bn, d), lambda i: (0, i, 0)),
                  pl.BlockSpec((bn, d), lambda i: (i, 0)),
                  pl.BlockSpec((2, bn, d), lambda i: (0, i, 0)),
                  pl.BlockSpec((1, d), lambda i: (0, 0)),
                  pl.BlockSpec(w2.shape, lambda i: (0, 0))],
        out_specs=pl.BlockSpec((bn, d), lambda i: (i, 0)),
        out_shape=jax.ShapeDtypeStruct((n, d), jnp.float32),
    )(acc_p, y1, deg_p, b1, w2)


def _tc_out(acc_p, y2, deg_p, b2, n):
    d = y2.shape[1]
    bn = 2000 if n % 2000 == 0 else n

    def body(a_ref, y_ref, dg_ref, b_ref, o_ref):
        dinv = _dinv_block(dg_ref[...])
        a = a_ref[0] + a_ref[1] + y_ref[...]
        o_ref[...] = dinv * a + b_ref[...]

    return pl.pallas_call(
        body,
        grid=(n // bn,),
        in_specs=[pl.BlockSpec((2, bn, d), lambda i: (0, i, 0)),
                  pl.BlockSpec((bn, d), lambda i: (i, 0)),
                  pl.BlockSpec((2, bn, d), lambda i: (0, i, 0)),
                  pl.BlockSpec((1, d), lambda i: (0, 0))],
        out_specs=pl.BlockSpec((bn, d), lambda i: (i, 0)),
        out_shape=jax.ShapeDtypeStruct((n, d), jnp.float32),
    )(acc_p, y2, deg_p, b2)


def kernel(x, edge_index, W1, b1, W2, b2):
    n = x.shape[0]
    e = edge_index.shape[1]
    src = edge_index[0].astype(jnp.int32)
    dst = edge_index[1].astype(jnp.int32)

    epw = -(-e // (NW * CH * 8)) * CH * 8
    nr = epw // CH
    e_pad = epw * NW
    pad = e_pad - e
    src_p = jnp.concatenate([src, jnp.zeros((pad,), jnp.int32)])
    dst_p = jnp.concatenate([dst, jnp.full((pad,), n, jnp.int32)])
    src2d = src_p.reshape(e_pad // CH, CH)
    dst2d = dst_p.reshape(e_pad // CH, CH)

    n_sh = -(-(n + 1) // (NS * 8)) * NS * 8
    zr = n_sh // NS
    zeros128 = jnp.zeros((zr, W1.shape[1]), jnp.float32)
    ones128 = jnp.ones((CH, W1.shape[1]), jnp.float32)
    b1r = b1.reshape(1, -1)
    b2r = b2.reshape(1, -1)

    deg_p = _sc_degree(dst2d, zeros128, ones128, n_sh, nr)
    xw1 = _tc_matmul(x, W1)
    y1 = _tc_scale(xw1, deg_p, n)
    acc1 = _sc_aggregate(y1, src2d, dst2d, zeros128, n_sh, nr)
    y2 = _tc_mid(acc1, y1, deg_p, b1r, W2, n)
    acc2 = _sc_aggregate(y2, src2d, dst2d, zeros128, n_sh, nr)
    return _tc_out(acc2, y2, deg_p, b2r, n)
